# Initial kernel scaffold; baseline (speedup 1.0000x reference)
import functools

import jax
import jax.numpy as jnp
from jax import lax
from jax.experimental import pallas as pl
from jax.experimental.pallas import tpu as pltpu
from jax.experimental.pallas import tpu_sc as plsc

_B = 16384 * 50
_D = 64
_NW = 32
_BPW = _B // _NW
_SUB = 128
_GROUP = 512
_JPG = _GROUP // _SUB
_NG = _BPW // _GROUP
_IDX_ROWS = _BPW // _SUB


def _body(idx_hbm, w_hbm, out_hbm, idx_all, rows0, rows1, sg0, sg1, so0, so1):
    wid = lax.axis_index("s") * 2 + lax.axis_index("c")
    base = wid * _BPW

    pltpu.sync_copy(idx_hbm.at[pl.ds(wid * _IDX_ROWS, _IDX_ROWS), :], idx_all)

    rows = (rows0, rows1)
    sg = (sg0, sg1)
    so = (so0, so1)

    def gather_cp(g, b, j):
        r = g * _JPG + j
        return pltpu.make_async_copy(
            w_hbm.at[idx_all.at[r]],
            rows[b].at[pl.ds(j * _SUB, _SUB), :],
            sg[b])

    def fire_gathers(g, b):
        for j in range(_JPG):
            gather_cp(g, b, j).start()

    def wait_gathers(g, b):
        for j in range(_JPG):
            gather_cp(g, b, j).wait()

    def write_block(g, b):
        cp = pltpu.make_async_copy(
            rows[b], out_hbm.at[pl.ds(base + g * _GROUP, _GROUP), :], so[b])
        cp.start()
        return cp

    for b in range(2):
        fire_gathers(b, b)

    def loop_body(go, carry):
        for b in range(2):
            g = 2 * go + b
            wait_gathers(g, b)
            cp = write_block(g, b)
            cp.wait()
            fire_gathers(g + 2, b)
        return carry

    lax.fori_loop(0, _NG // 2 - 1, loop_body, 0)

    for b in range(2):
        g = _NG - 2 + b
        wait_gathers(g, b)
        write_block(g, b).wait()


def kernel(x, weight):
    idx2d = x.reshape(_B // _SUB, _SUB)
    mesh = plsc.VectorSubcoreMesh(core_axis_name="c", subcore_axis_name="s")
    run = pl.kernel(
        _body,
        out_type=jax.ShapeDtypeStruct((_B, _D), jnp.float32),
        mesh=mesh,
        scratch_types=[
            pltpu.VMEM((_IDX_ROWS, _SUB), jnp.int32),
            pltpu.VMEM((_GROUP, _D), jnp.float32),
            pltpu.VMEM((_GROUP, _D), jnp.float32),
            pltpu.SemaphoreType.DMA,
            pltpu.SemaphoreType.DMA,
            pltpu.SemaphoreType.DMA,
            pltpu.SemaphoreType.DMA,
        ],
    )
    out = run(idx2d, weight)
    return out.reshape(x.shape[0], x.shape[1], _D)


# speedup vs baseline: 1.8733x; 1.8733x over previous
import functools

import jax
import jax.numpy as jnp
from jax import lax
from jax.experimental import pallas as pl
from jax.experimental.pallas import tpu as pltpu
from jax.experimental.pallas import tpu_sc as plsc

_B = 16384 * 50
_D = 64
_NW = 32
_BPW = _B // _NW
_SUB = 128
_GROUP = 512
_JPG = _GROUP // _SUB
_NG = _BPW // _GROUP
_IDX_ROWS = _BPW // _SUB


def _body(idx_hbm, w_hbm, out_hbm, idx_all, rows0, rows1, sg0, sg1, so0, so1):
    wid = lax.axis_index("s") * 2 + lax.axis_index("c")
    base = wid * _BPW

    pltpu.sync_copy(idx_hbm.at[pl.ds(wid * _IDX_ROWS, _IDX_ROWS), :], idx_all)

    rows = (rows0, rows1)
    sg = (sg0, sg1)
    so = (so0, so1)

    def gather_cp(g, b, j):
        r = g * _JPG + j
        return pltpu.make_async_copy(
            w_hbm.at[idx_all.at[r]],
            rows[b].at[pl.ds(j * _SUB, _SUB), :],
            sg[b])

    def fire_gathers(g, b):
        for j in range(_JPG):
            gather_cp(g, b, j).start()

    def wait_gathers(g, b):
        for j in range(_JPG):
            gather_cp(g, b, j).wait()

    def write_block(g, b):
        cp = pltpu.make_async_copy(
            rows[b], out_hbm.at[pl.ds(base + g * _GROUP, _GROUP), :], so[b])
        cp.start()
        return cp

    for b in range(2):
        fire_gathers(b, b)

    def loop_body(go, carry):
        for b in range(2):
            g = 2 * go + b
            wait_gathers(g, b)
            cp = write_block(g, b)
            cp.wait()
            fire_gathers(g + 2, b)
        return carry

    lax.fori_loop(0, _NG // 2 - 1, loop_body, 0)

    for b in range(2):
        g = _NG - 2 + b
        wait_gathers(g, b)
        write_block(g, b).wait()


def kernel(x, weight):
    idx2d = x.reshape(_B // _SUB, _SUB)
    mesh = plsc.VectorSubcoreMesh(core_axis_name="c", subcore_axis_name="s")
    run = pl.kernel(
        _body,
        out_type=jax.ShapeDtypeStruct((_B, _D), jnp.float32),
        mesh=mesh,
        scratch_types=[
            pltpu.VMEM((_IDX_ROWS, _SUB), jnp.int32),
            pltpu.VMEM((_GROUP, _D), jnp.float32),
            pltpu.VMEM((_GROUP, _D), jnp.float32),
            pltpu.SemaphoreType.DMA,
            pltpu.SemaphoreType.DMA,
            pltpu.SemaphoreType.DMA,
            pltpu.SemaphoreType.DMA,
        ],
        compiler_params=pltpu.CompilerParams(use_tc_tiling_on_sc=False),
    )
    out = run(idx2d, weight)
    return out.reshape(x.shape[0], x.shape[1], _D)


# speedup vs baseline: 1.8756x; 1.0012x over previous
import jax
import jax.numpy as jnp
from jax import lax
from jax.experimental import pallas as pl
from jax.experimental.pallas import tpu as pltpu
from jax.experimental.pallas import tpu_sc as plsc

_B = 16384 * 50
_D = 64
_NW = 32
_BPW = _B // _NW
_SUB = 128
_NG = _BPW // _SUB
_N = 8
_F = 4


def _body(idx_hbm, w_hbm, out_hbm, idx_all, rows, *sems):
    sg = sems[:_N]
    so = sems[_N:]
    wid = lax.axis_index("s") * 2 + lax.axis_index("c")
    base = wid * _BPW

    pltpu.sync_copy(idx_hbm.at[pl.ds(wid * _NG, _NG), :], idx_all)

    def gather_cp(k, s):
        return pltpu.make_async_copy(w_hbm.at[idx_all.at[k]], rows.at[s], sg[s])

    def write_cp(k, s):
        return pltpu.make_async_copy(
            rows.at[s], out_hbm.at[pl.ds(base + k * _SUB, _SUB), :], so[s])

    for s in range(_F):
        gather_cp(s, s).start()

    def loop_body(ko, carry):
        for s in range(_N):
            k = ko * _N + s
            gather_cp(k, s).wait()
            write_cp(k, s).start()
            sf = (s + _F) % _N

            @pl.when(k + _F < _NG)
            def _():
                @pl.when(k >= _N - _F)
                def _():
                    write_cp(k + _F - _N, sf).wait()
                gather_cp(k + _F, sf).start()
        return carry

    lax.fori_loop(0, _NG // _N, loop_body, 0)

    for s in range(_N):
        write_cp(_NG - _N + s, s).wait()


def kernel(x, weight):
    idx2d = x.reshape(_B // _SUB, _SUB)
    mesh = plsc.VectorSubcoreMesh(core_axis_name="c", subcore_axis_name="s")
    run = pl.kernel(
        _body,
        out_type=jax.ShapeDtypeStruct((_B, _D), jnp.float32),
        mesh=mesh,
        scratch_types=(
            [pltpu.VMEM((_NG, _SUB), jnp.int32),
             pltpu.VMEM((_N, _SUB, _D), jnp.float32)]
            + [pltpu.SemaphoreType.DMA] * (2 * _N)
        ),
        compiler_params=pltpu.CompilerParams(use_tc_tiling_on_sc=False),
    )
    out = run(idx2d, weight)
    return out.reshape(x.shape[0], x.shape[1], _D)
